# baseline (speedup 1.0000x reference)
import functools

import jax
import jax.numpy as jnp
from jax import lax
from jax.experimental import pallas as pl
from jax.experimental.pallas import tpu as pltpu
from jax.experimental.pallas import tpu_sc as plsc

_NUM_CORES = 2
_NUM_SUBCORES = 16
_NUM_WORKERS = _NUM_CORES * _NUM_SUBCORES

_NBUF = 2


def _emb_lookup(b, s, d, *, chunk):
    total = b * s
    b_per_w = total // _NUM_WORKERS
    n_chunks = b_per_w // chunk
    w_per_row = s // b_per_w

    mesh = plsc.VectorSubcoreMesh(core_axis_name="c", subcore_axis_name="s")

    @functools.partial(
        pl.kernel,
        mesh=mesh,
        out_type=jax.ShapeDtypeStruct((b, s, d), jnp.float32),
        scratch_types=[
            pltpu.VMEM((b_per_w,), jnp.int32),
            [pltpu.VMEM((chunk, d), jnp.float32) for _ in range(_NBUF)],
            pltpu.SemaphoreType.DMA,
            pltpu.SemaphoreType.DMA,
        ],
    )
    def emb(x_hbm, table_hbm, out_hbm, idx_v, bufs, gsem, osem):
        wid = lax.axis_index("s") * _NUM_CORES + lax.axis_index("c")
        row = wid // w_per_row
        col = (wid % w_per_row) * b_per_w
        pltpu.sync_copy(x_hbm.at[row, pl.ds(col, b_per_w)], idx_v)

        def gather(j, buf):
            return pltpu.async_copy(
                table_hbm.at[idx_v.at[pl.ds(j * chunk, chunk)]], buf, gsem
            )

        def write(j, buf):
            return pltpu.async_copy(
                buf, out_hbm.at[row, pl.ds(col + j * chunk, chunk)], osem
            )

        for j in range(_NBUF - 1):
            gather(j, bufs[j])
        pltpu.make_async_copy(
            table_hbm.at[idx_v.at[pl.ds(0, chunk)]], bufs[0], gsem
        ).wait()
        gather(_NBUF - 1, bufs[_NBUF - 1])
        write(0, bufs[0])

        n_steady = n_chunks - _NBUF
        @pl.loop(1, n_steady + 1, step=_NBUF)
        def _steady(j0):
            for t in range(_NBUF):
                j = j0 + t
                buf = bufs[(t + 1) % _NBUF]
                prev_buf = bufs[t % _NBUF]
                pltpu.make_async_copy(
                    table_hbm.at[idx_v.at[pl.ds(0, chunk)]], buf, gsem
                ).wait()
                pltpu.make_async_copy(
                    prev_buf, out_hbm.at[row, pl.ds(col, chunk)], osem
                ).wait()
                gather(j + _NBUF - 1, prev_buf)
                write(j, buf)

        for j in range(n_chunks - _NBUF + 1, n_chunks):
            buf = bufs[j % _NBUF]
            pltpu.make_async_copy(
                table_hbm.at[idx_v.at[pl.ds(0, chunk)]], buf, gsem
            ).wait()
            write(j, buf)
        for j in range(n_chunks - _NBUF, n_chunks):
            buf = bufs[j % _NBUF]
            pltpu.make_async_copy(
                buf, out_hbm.at[row, pl.ds(col, chunk)], osem
            ).wait()

    return emb


def kernel(x, table):
    b, s = x.shape
    d = table.shape[1]
    return _emb_lookup(b, s, d, chunk=64)(x, table)


# speedup vs baseline: 1.0222x; 1.0222x over previous
import functools

import jax
import jax.numpy as jnp
from jax import lax
from jax.experimental import pallas as pl
from jax.experimental.pallas import tpu as pltpu
from jax.experimental.pallas import tpu_sc as plsc

_NUM_CORES = 2
_NUM_SUBCORES = 16
_NUM_WORKERS = _NUM_CORES * _NUM_SUBCORES

_NBUF = 4


def _emb_lookup(b, s, d, *, chunk):
    total = b * s
    b_per_w = total // _NUM_WORKERS
    n_chunks = b_per_w // chunk
    w_per_row = s // b_per_w

    mesh = plsc.VectorSubcoreMesh(core_axis_name="c", subcore_axis_name="s")

    @functools.partial(
        pl.kernel,
        mesh=mesh,
        out_type=jax.ShapeDtypeStruct((b, s, d), jnp.float32),
        scratch_types=[
            pltpu.VMEM((b_per_w,), jnp.int32),
            [pltpu.VMEM((chunk, d), jnp.float32) for _ in range(_NBUF)],
            pltpu.SemaphoreType.DMA,
            pltpu.SemaphoreType.DMA,
            pltpu.SemaphoreType.DMA,
        ],
    )
    def emb(x_hbm, table_hbm, out_hbm, idx_v, bufs, gsem, osem, ssem):
        wid = lax.axis_index("s") * _NUM_CORES + lax.axis_index("c")
        row = wid // w_per_row
        col = (wid % w_per_row) * b_per_w
        head = _NBUF * chunk
        pltpu.sync_copy(x_hbm.at[row, pl.ds(col, head)], idx_v.at[pl.ds(0, head)])
        rest = pltpu.async_copy(
            x_hbm.at[row, pl.ds(col + head, b_per_w - head)],
            idx_v.at[pl.ds(head, b_per_w - head)],
            ssem,
        )

        def gather(j, buf):
            return pltpu.async_copy(
                table_hbm.at[idx_v.at[pl.ds(j * chunk, chunk)]], buf, gsem
            )

        def write(j, buf):
            return pltpu.async_copy(
                buf, out_hbm.at[row, pl.ds(col + j * chunk, chunk)], osem
            )

        for j in range(_NBUF - 1):
            gather(j, bufs[j])
        pltpu.make_async_copy(
            table_hbm.at[idx_v.at[pl.ds(0, chunk)]], bufs[0], gsem
        ).wait()
        gather(_NBUF - 1, bufs[_NBUF - 1])
        write(0, bufs[0])
        rest.wait()

        n_steady = n_chunks - _NBUF
        @pl.loop(1, n_steady + 1, step=_NBUF)
        def _steady(j0):
            for t in range(_NBUF):
                j = j0 + t
                buf = bufs[(t + 1) % _NBUF]
                prev_buf = bufs[t % _NBUF]
                pltpu.make_async_copy(
                    table_hbm.at[idx_v.at[pl.ds(0, chunk)]], buf, gsem
                ).wait()
                pltpu.make_async_copy(
                    prev_buf, out_hbm.at[row, pl.ds(col, chunk)], osem
                ).wait()
                gather(j + _NBUF - 1, prev_buf)
                write(j, buf)

        for j in range(n_chunks - _NBUF + 1, n_chunks):
            buf = bufs[j % _NBUF]
            pltpu.make_async_copy(
                table_hbm.at[idx_v.at[pl.ds(0, chunk)]], buf, gsem
            ).wait()
            write(j, buf)
        for j in range(n_chunks - _NBUF, n_chunks):
            buf = bufs[j % _NBUF]
            pltpu.make_async_copy(
                buf, out_hbm.at[row, pl.ds(col, chunk)], osem
            ).wait()

    return emb


def kernel(x, table):
    b, s = x.shape
    d = table.shape[1]
    return _emb_lookup(b, s, d, chunk=32)(x, table)
